# baseline (speedup 1.0000x reference)
import functools

import jax
import jax.numpy as jnp
from jax import lax
from jax.experimental import pallas as pl
from jax.experimental.pallas import tpu as pltpu
from jax.experimental.pallas import tpu_sc as plsc

B, S, D = 4096, 77, 512
SP = 80
MAIN = 72
TAIL = S - MAIN
NC, NS = 2, 16
NW = NC * NS
BPW = B // NW
LANES = 16


def _emb_body(x_hbm, tok_hbm, pos_hbm, out_hbm,
              pos_v, idx0, idx1, main0, main1, tail_v,
              isem0, isem1, gsem0, gsem1, tsem, osem0, osem1):
    wid = lax.axis_index("s") * NC + lax.axis_index("c")
    b0 = wid * BPW

    pltpu.sync_copy(pos_hbm, pos_v)

    idxs = (idx0, idx1)
    mains = (main0, main1)
    isems = (isem0, isem1)
    gsems = (gsem0, gsem1)
    osems = (osem0, osem1)

    def idx_src(c):
        return x_hbm.at[pl.ds((b0 + c) * SP, SP)]

    def start_idx(c, slot):
        pltpu.async_copy(idx_src(c), idxs[slot], isems[slot])

    def wait_idx(c, slot):
        pltpu.make_async_copy(idx_src(c), idxs[slot], isems[slot]).wait()

    def start_main_gather(c, slot):
        pltpu.async_copy(
            tok_hbm.at[idxs[slot].at[pl.ds(0, MAIN)]],
            mains[slot].at[pl.ds(0, MAIN)], gsems[slot])

    def wait_main_gather(c, slot):
        pltpu.make_async_copy(
            tok_hbm.at[idxs[slot].at[pl.ds(0, MAIN)]],
            mains[slot].at[pl.ds(0, MAIN)], gsems[slot]).wait()

    def start_tail_gather(c, slot):
        pltpu.async_copy(
            tok_hbm.at[idxs[slot].at[pl.ds(MAIN, SP - MAIN)]],
            tail_v, tsem)

    def wait_tail_gather(c, slot):
        pltpu.make_async_copy(
            tok_hbm.at[idxs[slot].at[pl.ds(MAIN, SP - MAIN)]],
            tail_v, tsem).wait()

    def start_out(c, slot):
        pltpu.async_copy(mains[slot], out_hbm.at[b0 + c], osems[slot])

    def wait_out(c, slot):
        pltpu.make_async_copy(mains[slot], out_hbm.at[b0 + c], osems[slot]).wait()

    start_idx(0, 0)
    start_idx(1, 1)
    wait_idx(0, 0)
    start_main_gather(0, 0)
    start_tail_gather(0, 0)

    def pair(p, carry):
        for sl_ in range(2):
            c = 2 * p + sl_
            slot, nslot = sl_, 1 - sl_

            @pl.when(c >= 1)
            def _():
                wait_out(c - 1, nslot)

            @pl.when(c + 1 < BPW)
            def _():
                wait_idx(c + 1, nslot)
                start_main_gather(c + 1, nslot)

            wait_main_gather(c, slot)
            wait_tail_gather(c, slot)

            main_v = mains[slot]

            for t in range(TAIL):
                for j in range(D // LANES):
                    dsl = pl.ds(j * LANES, LANES)
                    main_v[MAIN + t, dsl] = tail_v[t, dsl] + pos_v[MAIN + t, dsl]

            @pl.when(c + 1 < BPW)
            def _():
                start_tail_gather(c + 1, nslot)

            @pl.when(c + 2 < BPW)
            def _():
                start_idx(c + 2, slot)

            @plsc.parallel_loop(0, MAIN, unroll=4)
            def _(r):
                for j in range(D // LANES):
                    dsl = pl.ds(j * LANES, LANES)
                    main_v[r, dsl] = main_v[r, dsl] + pos_v[r, dsl]

            start_out(c, slot)
        return carry

    lax.fori_loop(0, BPW // 2, pair, 0)
    wait_out(BPW - 1, 1)


@functools.partial(
    pl.kernel,
    out_type=jax.ShapeDtypeStruct((B, S, D), jnp.float32),
    mesh=plsc.VectorSubcoreMesh(
        core_axis_name="c", subcore_axis_name="s", num_cores=NC, num_subcores=NS
    ),
    scratch_types=[
        pltpu.VMEM((S, D), jnp.float32),
        pltpu.VMEM((SP,), jnp.int32),
        pltpu.VMEM((SP,), jnp.int32),
        pltpu.VMEM((S, D), jnp.float32),
        pltpu.VMEM((S, D), jnp.float32),
        pltpu.VMEM((8, D), jnp.float32),
        pltpu.SemaphoreType.DMA,
        pltpu.SemaphoreType.DMA,
        pltpu.SemaphoreType.DMA,
        pltpu.SemaphoreType.DMA,
        pltpu.SemaphoreType.DMA,
        pltpu.SemaphoreType.DMA,
        pltpu.SemaphoreType.DMA,
    ],
)
def _emb(x_hbm, tok_hbm, pos_hbm, out_hbm, *rest):
    _emb_body(x_hbm, tok_hbm, pos_hbm, out_hbm, *rest)


def kernel(x, token_table, position_table):
    x_pad = jnp.pad(x.astype(jnp.int32), ((0, 0), (0, SP - S))).reshape(B * SP)
    return _emb(x_pad, token_table, position_table)
